# Initial kernel scaffold; baseline (speedup 1.0000x reference)
import jax
import jax.numpy as jnp
from jax.experimental import pallas as pl


def kernel(rna_data, tissue_id, tissue_table, seq_table, sec_table, loop_table, W1, b1, W2, b2, W3, b3):
    raise NotImplementedError("write your pallas kernel here")


# speedup vs baseline: 13.8309x; 13.8309x over previous
import jax
import jax.numpy as jnp
from jax.experimental import pallas as pl
from jax.experimental.pallas import tpu as pltpu

_B = 128
_D = 32
_NPOS = 1537
_PBLK = 53
_NSTEP = 29
_ROWS = _PBLK * _D
_TBL = 64
_H1 = 512
_H2 = 256


def _mlp_body(idx_ref, tbl_ref, w1_ref, b1_ref, w2_ref, b2_ref, w3_ref,
              b3_ref, out_ref, acc_ref, xt_ref):
    j = pl.program_id(0)

    idx = idx_ref[0]
    ids = jax.lax.broadcasted_iota(jnp.int32, (_TBL, _PBLK * _B), 0)
    oht = (ids == idx).astype(jnp.bfloat16)
    tblt = tbl_ref[...].astype(jnp.bfloat16)
    et = jax.lax.dot_general(
        tblt, oht, (((1,), (0,)), ((), ())),
        preferred_element_type=jnp.float32).astype(jnp.bfloat16)

    for g in range(_PBLK):
        xt_ref[g * _D:(g + 1) * _D, :] = et[:, g * _B:(g + 1) * _B]

    w1 = w1_ref[...].astype(jnp.bfloat16)
    part = jax.lax.dot_general(
        xt_ref[...], w1, (((0,), (0,)), ((), ())),
        preferred_element_type=jnp.float32)

    @pl.when(j == 0)
    def _():
        acc_ref[...] = jnp.zeros_like(acc_ref)

    acc_ref[...] += part

    @pl.when(j == _NSTEP - 1)
    def _():
        h1 = jnp.maximum(acc_ref[...] + b1_ref[...], 0.0)
        h2 = jax.lax.dot_general(
            h1.astype(jnp.bfloat16), w2_ref[...].astype(jnp.bfloat16),
            (((1,), (0,)), ((), ())), preferred_element_type=jnp.float32)
        h2 = jnp.maximum(h2 + b2_ref[...], 0.0)
        h3 = jax.lax.dot_general(
            h2.astype(jnp.bfloat16), w3_ref[...].astype(jnp.bfloat16),
            (((1,), (0,)), ((), ())), preferred_element_type=jnp.float32)
        out_ref[...] = h3 + b3_ref[...]


def kernel(rna_data, tissue_id, tissue_table, seq_table, sec_table,
           loop_table, W1, b1, W2, b2, W3, b3):
    tbl = jnp.zeros((_TBL, _D), jnp.float32)
    tbl = tbl.at[0:29].set(tissue_table)
    tbl = tbl.at[29:34].set(seq_table)
    tbl = tbl.at[34:38].set(sec_table)
    tbl = tbl.at[38:46].set(loop_table)
    tblt = tbl.T

    idxcols = jnp.concatenate([
        tissue_id[:, None],
        rna_data[:, :, 0] + 29,
        rna_data[:, :, 1] + 34,
        rna_data[:, :, 2] + 38,
    ], axis=1)
    idx = idxcols.T.reshape(_NSTEP, 1, _PBLK * _B)

    out = pl.pallas_call(
        _mlp_body,
        grid=(_NSTEP,),
        in_specs=[
            pl.BlockSpec((1, 1, _PBLK * _B), lambda j: (j, 0, 0)),
            pl.BlockSpec((_D, _TBL), lambda j: (0, 0)),
            pl.BlockSpec((_ROWS, _H1), lambda j: (j, 0)),
            pl.BlockSpec((1, _H1), lambda j: (0, 0)),
            pl.BlockSpec((_H1, _H2), lambda j: (0, 0)),
            pl.BlockSpec((1, _H2), lambda j: (0, 0)),
            pl.BlockSpec((_H2, 1), lambda j: (0, 0)),
            pl.BlockSpec((1, 1), lambda j: (0, 0)),
        ],
        out_specs=pl.BlockSpec((_B, 1), lambda j: (0, 0)),
        out_shape=jax.ShapeDtypeStruct((_B, 1), jnp.float32),
        scratch_shapes=[
            pltpu.VMEM((_B, _H1), jnp.float32),
            pltpu.VMEM((_ROWS, _B), jnp.bfloat16),
        ],
    )(idx, tblt, W1, b1.reshape(1, _H1), W2, b2.reshape(1, _H2), W3,
      b3.reshape(1, 1))
    return out


# speedup vs baseline: 17.2198x; 1.2450x over previous
import jax
import jax.numpy as jnp
from jax.experimental import pallas as pl
from jax.experimental.pallas import tpu as pltpu

_B = 128
_D = 32
_NPOS = 1537
_PBLK = 128
_NSTEP = 12
_NPOSP = 1664
_ROWS = _PBLK * _D
_TBL = 64
_H1 = 512
_H2 = 256


def _mlp_body(idx_ref, tbl_ref, w1_ref, w1t_ref, b1_ref, w2_ref, b2_ref,
              w3_ref, b3_ref, out_ref, acc_ref, xt_ref, idxt_ref):
    j = pl.program_id(0)

    @pl.when(j == 0)
    def _():
        acc_ref[...] = jnp.zeros_like(acc_ref)
        idxt_ref[...] = jnp.transpose(idx_ref[...], (1, 0))

    tblt = tbl_ref[...].astype(jnp.bfloat16)

    idxf = idxt_ref[pl.ds(j * _PBLK, _PBLK), :].reshape(1, _PBLK * _B)
    ids = jax.lax.broadcasted_iota(jnp.int32, (_TBL, _PBLK * _B), 0)
    oht = (ids == idxf).astype(jnp.bfloat16)
    et = jax.lax.dot_general(
        tblt, oht, (((1,), (0,)), ((), ())),
        preferred_element_type=jnp.float32).astype(jnp.bfloat16)

    for g in range(_PBLK):
        xt_ref[g * _D:(g + 1) * _D, :] = et[:, g * _B:(g + 1) * _B]

    w1 = w1_ref[...].astype(jnp.bfloat16)
    acc_ref[...] += jax.lax.dot_general(
        xt_ref[...], w1, (((0,), (0,)), ((), ())),
        preferred_element_type=jnp.float32)

    @pl.when(j == _NSTEP - 1)
    def _():
        ohtail = (jax.lax.broadcasted_iota(jnp.int32, (_TBL, _B), 0)
                  == idxt_ref[pl.ds(_NSTEP * _PBLK, 1), :]).astype(jnp.bfloat16)
        ettail = jax.lax.dot_general(
            tblt, ohtail, (((1,), (0,)), ((), ())),
            preferred_element_type=jnp.float32).astype(jnp.bfloat16)
        tail = jax.lax.dot_general(
            ettail, w1t_ref[...].astype(jnp.bfloat16),
            (((0,), (0,)), ((), ())), preferred_element_type=jnp.float32)
        h1 = jnp.maximum(acc_ref[...] + tail + b1_ref[...], 0.0)
        h2 = jax.lax.dot_general(
            h1.astype(jnp.bfloat16), w2_ref[...].astype(jnp.bfloat16),
            (((1,), (0,)), ((), ())), preferred_element_type=jnp.float32)
        h2 = jnp.maximum(h2 + b2_ref[...], 0.0)
        h3 = jax.lax.dot_general(
            h2.astype(jnp.bfloat16), w3_ref[...].astype(jnp.bfloat16),
            (((1,), (0,)), ((), ())), preferred_element_type=jnp.float32)
        out_ref[...] = h3 + b3_ref[...]


def kernel(rna_data, tissue_id, tissue_table, seq_table, sec_table,
           loop_table, W1, b1, W2, b2, W3, b3):
    tbl = jnp.zeros((_TBL, _D), jnp.float32)
    tbl = tbl.at[0:29].set(tissue_table)
    tbl = tbl.at[29:34].set(seq_table)
    tbl = tbl.at[34:38].set(sec_table)
    tbl = tbl.at[38:46].set(loop_table)
    tblt = tbl.T

    idxcols = jnp.concatenate([
        tissue_id[:, None],
        rna_data[:, :, 0] + 29,
        rna_data[:, :, 1] + 34,
        rna_data[:, :, 2] + 38,
        jnp.zeros((_B, _NPOSP - _NPOS), jnp.int32),
    ], axis=1)

    out = pl.pallas_call(
        _mlp_body,
        grid=(_NSTEP,),
        in_specs=[
            pl.BlockSpec((_B, _NPOSP), lambda j: (0, 0)),
            pl.BlockSpec((_D, _TBL), lambda j: (0, 0)),
            pl.BlockSpec((_ROWS, _H1), lambda j: (j, 0)),
            pl.BlockSpec((_D, _H1), lambda j: (_NSTEP * _PBLK, 0)),
            pl.BlockSpec((1, _H1), lambda j: (0, 0)),
            pl.BlockSpec((_H1, _H2), lambda j: (0, 0)),
            pl.BlockSpec((1, _H2), lambda j: (0, 0)),
            pl.BlockSpec((_H2, 1), lambda j: (0, 0)),
            pl.BlockSpec((1, 1), lambda j: (0, 0)),
        ],
        out_specs=pl.BlockSpec((_B, 1), lambda j: (0, 0)),
        out_shape=jax.ShapeDtypeStruct((_B, 1), jnp.float32),
        scratch_shapes=[
            pltpu.VMEM((_B, _H1), jnp.float32),
            pltpu.VMEM((_ROWS, _B), jnp.bfloat16),
            pltpu.VMEM((_NPOSP, _B), jnp.int32),
        ],
    )(idxcols, tblt, W1, W1, b1.reshape(1, _H1), W2, b2.reshape(1, _H2), W3,
      b3.reshape(1, 1))
    return out
